# baseline (speedup 1.0000x reference)
import functools

import jax
import jax.numpy as jnp
from jax import lax
from jax.experimental import pallas as pl
from jax.experimental.pallas import tpu as pltpu
from jax.experimental.pallas import tpu_sc as plsc

B = 4096
W = 20
NNEG = 5
NN = W * NNEG
D = 128
NSC = 4
NSUB = 2 * NSC
WPAD = 32
NPAD = 112

NCORES = 2
NSUBC = 16
NWORK = NCORES * NSUBC
RPW = B // NWORK
SB = 4
NSB = RPW // SB
CB = 4
NCH = RPW // CB


def _sc_dots_body(word_hbm, sub_hbm, cidx_hbm, pidx_hbm, ctxidx_hbm,
                  nidx_hbm, dctx_hbm, dnoise_hbm,
                  cidx_v, pidx_v, ctxidx_v, nidx_v, sb0, sb1, tgts_v,
                  cb0, cb1, nb0, nb1, dctx_v, dnoise_v,
                  ssc0, ssc1, ssp0, ssp1, sc0, sc1, sn0, sn1):
    wid = lax.axis_index("s") * NCORES + lax.axis_index("c")
    base = wid * RPW
    pltpu.sync_copy(
        cidx_hbm.at[pl.ds(pl.multiple_of(base * NSC, 8), RPW * NSC)], cidx_v)
    pltpu.sync_copy(
        pidx_hbm.at[pl.ds(pl.multiple_of(base * NSC, 8), RPW * NSC)], pidx_v)
    pltpu.sync_copy(
        ctxidx_hbm.at[pl.ds(pl.multiple_of(base * W, 8), RPW * W)], ctxidx_v)
    pltpu.sync_copy(
        nidx_hbm.at[pl.ds(pl.multiple_of(base * NN, 8), RPW * NN)], nidx_v)

    lanes = lax.broadcasted_iota(jnp.int32, (16,), 0)

    def sgather(b, buf, semc, semp):
        so = pl.multiple_of(b * SB * NSC, 8)
        pltpu.async_copy(sub_hbm.at[cidx_v.at[pl.ds(so, SB * NSC)]],
                         buf.at[pl.ds(0, SB * NSC)], semc)
        pltpu.async_copy(sub_hbm.at[pidx_v.at[pl.ds(so, SB * NSC)]],
                         buf.at[pl.ds(SB * NSC, SB * NSC)], semp)

    def swait(buf, semc, semp):
        pltpu.make_async_copy(sub_hbm.at[pl.ds(0, SB * NSC)],
                              buf.at[pl.ds(0, SB * NSC)], semc).wait()
        pltpu.make_async_copy(sub_hbm.at[pl.ds(0, SB * NSC)],
                              buf.at[pl.ds(SB * NSC, SB * NSC)], semp).wait()

    def wgather(c, cbuf, nbuf, semc, semn):
        co = pl.multiple_of(c * CB * W, 8)
        no = pl.multiple_of(c * CB * NN, 8)
        pltpu.async_copy(word_hbm.at[ctxidx_v.at[pl.ds(co, CB * W)]],
                         cbuf.at[pl.ds(0, CB * W)], semc)
        pltpu.async_copy(word_hbm.at[nidx_v.at[pl.ds(no, CB * NN)]],
                         nbuf.at[pl.ds(0, CB * NN)], semn)

    def wwait(cbuf, nbuf, semc, semn):
        pltpu.make_async_copy(word_hbm.at[pl.ds(0, CB * W)],
                              cbuf.at[pl.ds(0, CB * W)], semc).wait()
        pltpu.make_async_copy(word_hbm.at[pl.ds(0, CB * NN)],
                              nbuf.at[pl.ds(0, CB * NN)], semn).wait()

    wgather(0, cb0, nb0, sc0, sn0)
    sgather(0, sb0, ssc0, ssp0)

    def ldbf(buf, r, q):
        return plsc.bitcast(buf[r, pl.ds(16 * q, 16)], jnp.bfloat16)

    def pool(b, buf):
        for i in range(SB):
            row = b * SB + i
            t = [ldbf(buf, i * NSC, q) for q in range(4)]
            for r in range(1, NSC):
                for q in range(4):
                    t[q] = t[q] + ldbf(buf, i * NSC + r, q)
            for r in range(NSC):
                for q in range(4):
                    t[q] = t[q] + ldbf(buf, SB * NSC + i * NSC + r, q)
            for q in range(4):
                tgts_v[row, pl.ds(32 * q, 32)] = t[q] * jnp.bfloat16(1.0 / NSUB)

    def p1body(bi, carry):
        b0 = bi * 2
        sgather(b0 + 1, sb1, ssc1, ssp1)
        swait(sb0, ssc0, ssp0)
        pool(b0, sb0)

        @pl.when(bi + 1 < NSB // 2)
        def _():
            sgather(b0 + 2, sb0, ssc0, ssp0)

        swait(sb1, ssc1, ssp1)
        pool(b0 + 1, sb1)
        return carry

    lax.fori_loop(0, NSB // 2, p1body, 0)

    def dot16(t, buf, rbase):
        vals = jnp.zeros((16,), jnp.float32)
        for l in range(16):
            r = rbase + l
            p0 = t[0] * ldbf(buf, r, 0)
            p1 = t[1] * ldbf(buf, r, 1)
            p2 = t[2] * ldbf(buf, r, 2)
            p3 = t[3] * ldbf(buf, r, 3)
            ps = (p0 + p1) + (p2 + p3)
            a, bb = plsc.unpack(ps, format=plsc.PackFormat.INTERLEAVED)
            vals = jnp.where(lanes == l, jnp.sum(a + bb), vals)
        return vals

    def compute(c, cbuf, nbuf):
        for i in range(CB):
            brow = c * CB + i
            t = [tgts_v[brow, pl.ds(32 * q, 32)] for q in range(4)]

            def cgrp(g, carry):
                dctx_v[brow, pl.ds(g * 16, 16)] = dot16(t, cbuf, i * W + g * 16)
                return carry

            lax.fori_loop(0, WPAD // 16, cgrp, 0)

            def ngrp(g, carry):
                dnoise_v[brow, pl.ds(g * 16, 16)] = dot16(t, nbuf,
                                                          i * NN + g * 16)
                return carry

            lax.fori_loop(0, NPAD // 16, ngrp, 0)

    def body(ci, carry):
        c0 = ci * 2
        wgather(c0 + 1, cb1, nb1, sc1, sn1)
        wwait(cb0, nb0, sc0, sn0)
        compute(c0, cb0, nb0)

        @pl.when(ci + 1 < NCH // 2)
        def _():
            wgather(c0 + 2, cb0, nb0, sc0, sn0)

        wwait(cb1, nb1, sc1, sn1)
        compute(c0 + 1, cb1, nb1)
        return carry

    lax.fori_loop(0, NCH // 2, body, 0)
    pltpu.sync_copy(dctx_v, dctx_hbm.at[pl.ds(pl.multiple_of(base, 8), RPW)])
    pltpu.sync_copy(dnoise_v,
                    dnoise_hbm.at[pl.ds(pl.multiple_of(base, 8), RPW)])


@functools.lru_cache(maxsize=1)
def _sc_dots():
    return pl.kernel(
        _sc_dots_body,
        mesh=plsc.VectorSubcoreMesh(core_axis_name="c", subcore_axis_name="s"),
        compiler_params=pltpu.CompilerParams(
            needs_layout_passes=False, use_tc_tiling_on_sc=False),
        out_type=(jax.ShapeDtypeStruct((B, WPAD), jnp.float32),
                  jax.ShapeDtypeStruct((B, NPAD), jnp.float32)),
        scratch_types=[
            pltpu.VMEM((RPW * NSC,), jnp.int32),
            pltpu.VMEM((RPW * NSC,), jnp.int32),
            pltpu.VMEM((RPW * W,), jnp.int32),
            pltpu.VMEM((RPW * NN,), jnp.int32),
            pltpu.VMEM((2 * SB * NSC, D // 2), jnp.int32),
            pltpu.VMEM((2 * SB * NSC, D // 2), jnp.int32),
            pltpu.VMEM((RPW, D), jnp.bfloat16),
            pltpu.VMEM((CB * W + 16, D // 2), jnp.int32),
            pltpu.VMEM((CB * W + 16, D // 2), jnp.int32),
            pltpu.VMEM((CB * NN + 16, D // 2), jnp.int32),
            pltpu.VMEM((CB * NN + 16, D // 2), jnp.int32),
            pltpu.VMEM((RPW, WPAD), jnp.float32),
            pltpu.VMEM((RPW, NPAD), jnp.float32),
        ] + [pltpu.SemaphoreType.DMA] * 8,
    )


def _tc_loss_body(dctx_ref, dnoise_ref, ctxidx_ref, out_ref):
    mask = (ctxidx_ref[...] >= 2).astype(jnp.float32)
    colc = lax.broadcasted_iota(jnp.int32, (B, WPAD), 1)
    coln = lax.broadcasted_iota(jnp.int32, (B, NPAD), 1)
    dc = jnp.where(colc < W, dctx_ref[...], 0.0)
    dn = jnp.where(coln < NN, dnoise_ref[...], 0.0)
    val_c = jnp.log(1.0 / (1.0 + jnp.exp(-dc)) + 1e-5)
    val_n = jnp.log(1.0 / (1.0 + jnp.exp(dn)) + 1e-5)
    cw = lax.broadcasted_iota(jnp.int32, (W, WPAD), 1)
    cr = lax.broadcasted_iota(jnp.int32, (W, WPAD), 0)
    ec = ((cw == cr) & (cw < W)).astype(jnp.float32)
    nw = lax.broadcasted_iota(jnp.int32, (W, NPAD), 1)
    nr = lax.broadcasted_iota(jnp.int32, (W, NPAD), 0)
    en = ((nw // NNEG == nr) & (nw < NN)).astype(jnp.float32)
    mask_c = jnp.dot(mask, ec, preferred_element_type=jnp.float32)
    mask_n = jnp.dot(mask, en, preferred_element_type=jnp.float32)
    total = jnp.sum(val_c * mask_c) + jnp.sum(val_n * mask_n)
    out_ref[0, 0] = -total / B


def kernel(word_emb, char_emb, compo_emb, tgt_compo_idx, tgt_char_idx,
           ctx_word_idx, noise_idx):
    nchar = char_emb.shape[0]

    def pack(tab):
        b16 = tab.astype(jnp.bfloat16)
        lo = lax.bitcast_convert_type(b16[:, :D // 2], jnp.uint16)
        hi = lax.bitcast_convert_type(b16[:, D // 2:], jnp.uint16)
        w = lo.astype(jnp.uint32) | (hi.astype(jnp.uint32) << 16)
        return lax.bitcast_convert_type(w, jnp.int32)

    word_tab = pack(word_emb)
    sub_tab = pack(jnp.concatenate([char_emb, compo_emb], axis=0))

    cidx = tgt_char_idx.astype(jnp.int32).reshape(-1)
    pidx = (tgt_compo_idx.astype(jnp.int32) + nchar).reshape(-1)
    ctxidx = ctx_word_idx.astype(jnp.int32).reshape(-1)
    nidx = noise_idx.astype(jnp.int32).reshape(-1)

    dctx, dnoise = _sc_dots()(word_tab, sub_tab, cidx, pidx, ctxidx, nidx)

    loss = pl.pallas_call(
        _tc_loss_body,
        out_shape=jax.ShapeDtypeStruct((1, 1), jnp.float32),
        out_specs=pl.BlockSpec(memory_space=pltpu.SMEM),
    )(dctx, dnoise, ctx_word_idx.astype(jnp.int32))
    return loss[0, 0]


# speedup vs baseline: 1.5211x; 1.5211x over previous
import functools

import jax
import jax.numpy as jnp
from jax import lax
from jax.experimental import pallas as pl
from jax.experimental.pallas import tpu as pltpu
from jax.experimental.pallas import tpu_sc as plsc

B = 4096
W = 20
NNEG = 5
D = 128
NSUB = 8
NWN = W * (1 + NNEG)

NCORES = 2
NSUBC = 16
NWORK = NCORES * NSUBC
RPW = B // NWORK
SB = 4
NSB = RPW // SB
CB = 2
NCH = RPW // CB
NGRP = (NWN + 15) // 16


def _sc_dots_body(word_hbm, sub_hbm, subidx_hbm, wnidx_hbm, out_hbm,
                  subidx_v, wnidx_v, sb0, sb1, tgts_v, wn0, wn1, out_v,
                  ss0, ss1, sw0, sw1):
    wid = lax.axis_index("s") * NCORES + lax.axis_index("c")
    base = wid * RPW
    pltpu.sync_copy(
        subidx_hbm.at[pl.ds(pl.multiple_of(base * NSUB, 8), RPW * NSUB)],
        subidx_v)
    pltpu.sync_copy(
        wnidx_hbm.at[pl.ds(pl.multiple_of(base * NWN, 8), RPW * NWN)],
        wnidx_v)

    lanes = lax.broadcasted_iota(jnp.int32, (16,), 0)

    def wgather(c, buf, sem):
        wo = pl.multiple_of(c * CB * NWN, 8)
        pltpu.async_copy(word_hbm.at[wnidx_v.at[pl.ds(wo, CB * NWN)]],
                         buf.at[pl.ds(0, CB * NWN)], sem)

    def wwait(buf, sem):
        pltpu.make_async_copy(word_hbm.at[pl.ds(0, CB * NWN)],
                              buf.at[pl.ds(0, CB * NWN)], sem).wait()

    def sgather(b, buf, sem):
        so = pl.multiple_of(b * SB * NSUB, 8)
        pltpu.async_copy(sub_hbm.at[subidx_v.at[pl.ds(so, SB * NSUB)]],
                         buf, sem)

    def swait(buf, sem):
        pltpu.make_async_copy(sub_hbm.at[pl.ds(0, SB * NSUB)], buf,
                              sem).wait()

    wgather(0, wn0, sw0)
    sgather(0, sb0, ss0)

    def pool(b, buf):
        for i in range(SB):
            row = b * SB + i
            t = [buf[i * NSUB, pl.ds(16 * k, 16)] for k in range(8)]
            for r in range(1, NSUB):
                for k in range(8):
                    t[k] = t[k] + buf[i * NSUB + r, pl.ds(16 * k, 16)]
            for k in range(8):
                tgts_v[row, pl.ds(16 * k, 16)] = t[k] * (1.0 / NSUB)

    def p1body(bi, carry):
        b0 = bi * 2
        sgather(b0 + 1, sb1, ss1)
        swait(sb0, ss0)
        pool(b0, sb0)

        @pl.when(bi + 1 < NSB // 2)
        def _():
            sgather(b0 + 2, sb0, ss0)

        swait(sb1, ss1)
        pool(b0 + 1, sb1)
        return carry

    lax.fori_loop(0, NSB // 2, p1body, 0)

    def dot16(t, buf, rbase):
        chains = [jnp.zeros((16,), jnp.float32) for _ in range(4)]
        for l in range(16):
            r = rbase + l
            acc = t[0] * buf[r, pl.ds(0, 16)]
            for k in range(1, 8):
                acc = acc + t[k] * buf[r, pl.ds(16 * k, 16)]
            c = l % 4
            chains[c] = jnp.where(lanes == l, jnp.sum(acc), chains[c])
        return (chains[0] + chains[1]) + (chains[2] + chains[3])

    def compute(c, buf):
        for i in range(CB):
            brow = c * CB + i
            t = [tgts_v[brow, pl.ds(16 * k, 16)] for k in range(8)]
            obase = pl.multiple_of(brow * NWN, 8)

            def grp(g2, carry):
                for u in range(2):
                    g = g2 * 2 + u
                    out_v[pl.ds(obase + g * 16, 16)] = dot16(
                        t, buf, i * NWN + g * 16)
                return carry

            lax.fori_loop(0, NGRP // 2, grp, 0)

    def body(ci, carry):
        c0 = ci * 2
        wgather(c0 + 1, wn1, sw1)
        wwait(wn0, sw0)
        compute(c0, wn0)

        @pl.when(ci + 1 < NCH // 2)
        def _():
            wgather(c0 + 2, wn0, sw0)

        wwait(wn1, sw1)
        compute(c0 + 1, wn1)
        return carry

    lax.fori_loop(0, NCH // 2, body, 0)
    pltpu.sync_copy(
        out_v.at[pl.ds(0, RPW * NWN)],
        out_hbm.at[pl.ds(pl.multiple_of(base * NWN, 8), RPW * NWN)])


@functools.lru_cache(maxsize=1)
def _sc_dots():
    return pl.kernel(
        _sc_dots_body,
        mesh=plsc.VectorSubcoreMesh(core_axis_name="c", subcore_axis_name="s"),
        compiler_params=pltpu.CompilerParams(
            needs_layout_passes=False, use_tc_tiling_on_sc=False),
        out_type=jax.ShapeDtypeStruct((B * NWN,), jnp.float32),
        scratch_types=[
            pltpu.VMEM((RPW * NSUB,), jnp.int32),
            pltpu.VMEM((RPW * NWN,), jnp.int32),
            pltpu.VMEM((SB * NSUB, D), jnp.float32),
            pltpu.VMEM((SB * NSUB, D), jnp.float32),
            pltpu.VMEM((RPW, D), jnp.float32),
            pltpu.VMEM((CB * NWN + 16, D), jnp.float32),
            pltpu.VMEM((CB * NWN + 16, D), jnp.float32),
            pltpu.VMEM((RPW * NWN + 16,), jnp.float32),
            pltpu.SemaphoreType.DMA,
            pltpu.SemaphoreType.DMA,
            pltpu.SemaphoreType.DMA,
            pltpu.SemaphoreType.DMA,
        ],
    )


def _tc_loss_body(dots_ref, ctxidx_ref, out_ref):
    dots = dots_ref[...]
    mask = (ctxidx_ref[...] >= 2).astype(jnp.float32)
    col = lax.broadcasted_iota(jnp.int32, (B, NWN), 1)
    is_ctx = col < W
    sig_pos = 1.0 / (1.0 + jnp.exp(-dots))
    sig_neg = 1.0 / (1.0 + jnp.exp(dots))
    val = jnp.where(is_ctx,
                    jnp.log(sig_pos + 1e-5),
                    jnp.log(sig_neg + 1e-5))
    colw = lax.broadcasted_iota(jnp.int32, (W, NWN), 1)
    roww = lax.broadcasted_iota(jnp.int32, (W, NWN), 0)
    src = jnp.where(colw < W, colw, (colw - W) // NNEG)
    expand = (src == roww).astype(jnp.float32)
    mask_full = jnp.dot(mask, expand, preferred_element_type=jnp.float32)
    out_ref[0, 0] = -jnp.sum(val * mask_full) / B


def kernel(word_emb, char_emb, compo_emb, tgt_compo_idx, tgt_char_idx,
           ctx_word_idx, noise_idx):
    nchar = char_emb.shape[0]
    sub_tab = jnp.concatenate([char_emb, compo_emb], axis=0)
    sub_idx = jnp.concatenate(
        [tgt_char_idx.astype(jnp.int32),
         tgt_compo_idx.astype(jnp.int32) + nchar], axis=1).reshape(-1)
    wn_idx = jnp.concatenate(
        [ctx_word_idx.astype(jnp.int32),
         noise_idx.astype(jnp.int32)], axis=1).reshape(-1)

    dots = _sc_dots()(word_emb, sub_tab, sub_idx, wn_idx)

    loss = pl.pallas_call(
        _tc_loss_body,
        out_shape=jax.ShapeDtypeStruct((1, 1), jnp.float32),
        out_specs=pl.BlockSpec(memory_space=pltpu.SMEM),
    )(dots.reshape(B, NWN), ctx_word_idx.astype(jnp.int32))
    return loss[0, 0]
